# Initial kernel scaffold; baseline (speedup 1.0000x reference)
import jax
import jax.numpy as jnp
from jax.experimental import pallas as pl


def kernel(atomic_numbers, table):
    raise NotImplementedError("write your pallas kernel here")


# speedup vs baseline: 1.7003x; 1.7003x over previous
import functools

import jax
import jax.numpy as jnp
from jax import lax
from jax.experimental import pallas as pl
from jax.experimental.pallas import tpu as pltpu
from jax.experimental.pallas import tpu_sc as plsc

NUM_ELEMENTS = 120
EMBED_DIM = 64
N_ATOMS = 100000

NC = 2
NS = 16
NW = NC * NS

PER_W = N_ATOMS // NW
CHUNK = 128
FULL_CHUNKS = PER_W // CHUNK
TAIL = PER_W - FULL_CHUNKS * CHUNK
PER_W_PAD = ((PER_W + 7) // 8) * 8


def _gather_body(table_hbm, idx_hbm, out_hbm, idx_v, rows_v, tail_v, gsem):
    wid = lax.axis_index("s") * NC + lax.axis_index("c")
    base = wid * PER_W
    pltpu.sync_copy(idx_hbm.at[wid], idx_v)

    def chunk(j, carry):
        pltpu.async_copy(
            table_hbm.at[idx_v.at[pl.ds(j * CHUNK, CHUNK)]], rows_v, gsem
        ).wait()
        pltpu.sync_copy(rows_v, out_hbm.at[pl.ds(base + j * CHUNK, CHUNK)])
        return carry

    lax.fori_loop(0, FULL_CHUNKS, chunk, 0, unroll=False)

    pltpu.async_copy(
        table_hbm.at[idx_v.at[pl.ds(FULL_CHUNKS * CHUNK, TAIL)]], tail_v, gsem
    ).wait()
    pltpu.sync_copy(
        tail_v, out_hbm.at[pl.ds(base + FULL_CHUNKS * CHUNK, TAIL)]
    )


@jax.jit
def _sc_gather(table, idx_pad):
    mesh = plsc.VectorSubcoreMesh(core_axis_name="c", subcore_axis_name="s")
    f = functools.partial(
        pl.kernel,
        out_type=jax.ShapeDtypeStruct((N_ATOMS, EMBED_DIM), jnp.float32),
        mesh=mesh,
        scratch_types=[
            pltpu.VMEM((PER_W_PAD,), jnp.int32),
            pltpu.VMEM((CHUNK, EMBED_DIM), jnp.float32),
            pltpu.VMEM((TAIL, EMBED_DIM), jnp.float32),
            pltpu.SemaphoreType.DMA,
        ],
        compiler_params=pltpu.CompilerParams(use_tc_tiling_on_sc=False),
    )(_gather_body)
    return f(table, idx_pad)


def kernel(atomic_numbers, table):
    idx = atomic_numbers.astype(jnp.int32).reshape(NW, PER_W)
    idx_pad = jnp.pad(idx, ((0, 0), (0, PER_W_PAD - PER_W)))
    return _sc_gather(table, idx_pad)
